# baseline (speedup 1.0000x reference)
import jax
import jax.numpy as jnp
from jax import lax
from jax.experimental import pallas as pl
from jax.experimental.pallas import tpu as pltpu
from jax.experimental.pallas import tpu_sc as plsc

NC = 2
NS = 16
NW = NC * NS

VOCAB = 194
DIM = 128
BATCH = 4096
SEQ = 200

B_TOTAL = BATCH * SEQ
B_SC = 19 * 32768
B_PER_W = B_SC // NW
CHUNK = 128
N_CHUNKS = B_PER_W // CHUNK
NBUF = 6
G = 3

_STEADY_LO = NBUF - G
_N_GROUPS = (N_CHUNKS - NBUF) // NBUF
_STEADY_HI = _STEADY_LO + _N_GROUPS * NBUF

B_TC = B_TOTAL - B_SC
TC_BLOCK = 256
VPAD = 256
N_TC_BLOCKS = B_TC // TC_BLOCK
assert N_TC_BLOCKS * TC_BLOCK == B_TC


def _sc_body(table_hbm, idx_hbm, out_hbm, idx_v, *rest):
    rows = rest[:NBUF]
    table_sh = rest[NBUF]
    gsem = rest[NBUF + 1:2 * NBUF + 1]
    osem = rest[2 * NBUF + 1:]

    cid = lax.axis_index("c")
    sid = lax.axis_index("s")
    wid = sid * NC + cid
    out_base = wid * B_PER_W

    @pl.when(sid == 0)
    def _():
        pltpu.sync_copy(table_hbm, table_sh)

    plsc.subcore_barrier()

    pltpu.sync_copy(idx_hbm.at[pl.ds(wid * N_CHUNKS, N_CHUNKS)], idx_v)

    def start_gather(j, b):
        pltpu.make_async_copy(table_sh.at[idx_v.at[j]], rows[b], gsem[b]).start()

    def wait_gather(j, b):
        pltpu.make_async_copy(table_sh.at[idx_v.at[j]], rows[b], gsem[b]).wait()

    def start_out(j, b):
        pltpu.make_async_copy(
            rows[b], out_hbm.at[pl.ds(out_base + j * CHUNK, CHUNK)], osem[b]
        ).start()

    def wait_out(j, b):
        pltpu.make_async_copy(
            rows[b], out_hbm.at[pl.ds(out_base + j * CHUNK, CHUNK)], osem[b]
        ).wait()

    for b in range(G):
        start_gather(b, b)
    for j in range(_STEADY_LO):
        wait_gather(j, j)
        start_out(j, j)
        start_gather(j + G, (j + G) % NBUF)

    def group(g, _):
        j0 = _STEADY_LO + g * NBUF
        for b in range(NBUF):
            j = j0 + b
            wb = (_STEADY_LO + b) % NBUF
            wait_out(j - G, b)
            start_gather(j + G, b)
            wait_gather(j, wb)
            start_out(j, wb)
        return 0

    lax.fori_loop(0, _N_GROUPS, group, 0)

    for j in range(_STEADY_HI, N_CHUNKS):
        if j + G < N_CHUNKS:
            wait_out(j - G, (j + G) % NBUF)
            start_gather(j + G, (j + G) % NBUF)
        wait_gather(j, j % NBUF)
        start_out(j, j % NBUF)
    for j in range(N_CHUNKS - NBUF, N_CHUNKS):
        wait_out(j, j % NBUF)


def _sc_part(idx_flat, table):
    idx2d = idx_flat.reshape(NW * N_CHUNKS, CHUNK)
    mesh = plsc.VectorSubcoreMesh(core_axis_name="c", subcore_axis_name="s")
    return pl.kernel(
        _sc_body,
        out_type=jax.ShapeDtypeStruct((B_SC, DIM), jnp.float32),
        mesh=mesh,
        scratch_types=[
            pltpu.VMEM((N_CHUNKS, CHUNK), jnp.int32),
            *[pltpu.VMEM((CHUNK, DIM), jnp.float32) for _ in range(NBUF)],
            pltpu.VMEM_SHARED((VOCAB, DIM), jnp.float32),
            *[pltpu.SemaphoreType.DMA for _ in range(2 * NBUF)],
        ],
    )(table, idx2d)


def _tc_body(idx_ref, table_ref, out_ref):
    idx = idx_ref[0]
    onehot = jnp.where(
        lax.broadcasted_iota(jnp.int32, (TC_BLOCK, VPAD), 1)
        == idx.reshape(TC_BLOCK, 1),
        1.0,
        0.0,
    ).astype(jnp.float32)
    out_ref[...] = jnp.dot(
        onehot, table_ref[...], preferred_element_type=jnp.float32
    )


def _tc_part(idx_flat, table):
    idx3d = idx_flat.reshape(N_TC_BLOCKS, 1, TC_BLOCK)
    table_pad = jnp.pad(table, ((0, VPAD - VOCAB), (0, 0)))
    return pl.pallas_call(
        _tc_body,
        grid=(N_TC_BLOCKS,),
        in_specs=[
            pl.BlockSpec((1, 1, TC_BLOCK), lambda i: (i, 0, 0)),
            pl.BlockSpec((VPAD, DIM), lambda i: (0, 0)),
        ],
        out_specs=pl.BlockSpec((TC_BLOCK, DIM), lambda i: (i, 0)),
        out_shape=jax.ShapeDtypeStruct((B_TC, DIM), jnp.float32),
    )(idx3d, table_pad)


def kernel(substructure_indices, embedding_table):
    idx = substructure_indices.astype(jnp.int32).reshape(-1)
    out_sc = _sc_part(idx[:B_SC], embedding_table)
    out_tc = _tc_part(idx[B_SC:], embedding_table)
    return jnp.concatenate([out_sc, out_tc], axis=0).reshape(BATCH, SEQ, DIM)


# speedup vs baseline: 1.5018x; 1.5018x over previous
import jax
import jax.numpy as jnp
from jax import lax
from jax.experimental import pallas as pl
from jax.experimental.pallas import tpu as pltpu
from jax.experimental.pallas import tpu_sc as plsc

NC = 2
NS = 16
NW = NC * NS

VOCAB = 194
DIM = 128
BATCH = 4096
SEQ = 200

B_TOTAL = BATCH * SEQ
B_SC = 19 * 32768
B_PER_W = B_SC // NW
CHUNK = 128
N_CHUNKS = B_PER_W // CHUNK
NBUF = 6
G = 3

_STEADY_LO = NBUF - G
_N_GROUPS = (N_CHUNKS - NBUF) // NBUF
_STEADY_HI = _STEADY_LO + _N_GROUPS * NBUF

B_TC = B_TOTAL - B_SC
TC_BLOCK = 2048
VPAD = 256
N_TC_BLOCKS = B_TC // TC_BLOCK
assert N_TC_BLOCKS * TC_BLOCK == B_TC


def _sc_body(table_hbm, idx_hbm, out_hbm, idx_v, *rest):
    rows = rest[:NBUF]
    table_sh = rest[NBUF]
    gsem = rest[NBUF + 1:2 * NBUF + 1]
    osem = rest[2 * NBUF + 1:]

    cid = lax.axis_index("c")
    sid = lax.axis_index("s")
    wid = sid * NC + cid
    out_base = wid * B_PER_W

    @pl.when(sid == 0)
    def _():
        pltpu.sync_copy(table_hbm, table_sh)

    plsc.subcore_barrier()

    pltpu.sync_copy(idx_hbm.at[pl.ds(wid * N_CHUNKS, N_CHUNKS)], idx_v)

    def start_gather(j, b):
        pltpu.make_async_copy(table_sh.at[idx_v.at[j]], rows[b], gsem[b]).start()

    def wait_gather(j, b):
        pltpu.make_async_copy(table_sh.at[idx_v.at[j]], rows[b], gsem[b]).wait()

    def start_out(j, b):
        pltpu.make_async_copy(
            rows[b], out_hbm.at[pl.ds(out_base + j * CHUNK, CHUNK)], osem[b]
        ).start()

    def wait_out(j, b):
        pltpu.make_async_copy(
            rows[b], out_hbm.at[pl.ds(out_base + j * CHUNK, CHUNK)], osem[b]
        ).wait()

    for b in range(G):
        start_gather(b, b)
    for j in range(_STEADY_LO):
        wait_gather(j, j)
        start_out(j, j)
        start_gather(j + G, (j + G) % NBUF)

    def group(g, _):
        j0 = _STEADY_LO + g * NBUF
        for b in range(NBUF):
            j = j0 + b
            wb = (_STEADY_LO + b) % NBUF
            wait_out(j - G, b)
            start_gather(j + G, b)
            wait_gather(j, wb)
            start_out(j, wb)
        return 0

    lax.fori_loop(0, _N_GROUPS, group, 0)

    for j in range(_STEADY_HI, N_CHUNKS):
        if j + G < N_CHUNKS:
            wait_out(j - G, (j + G) % NBUF)
            start_gather(j + G, (j + G) % NBUF)
        wait_gather(j, j % NBUF)
        start_out(j, j % NBUF)
    for j in range(N_CHUNKS - NBUF, N_CHUNKS):
        wait_out(j, j % NBUF)


def _sc_part(idx_flat, table):
    idx2d = idx_flat.reshape(NW * N_CHUNKS, CHUNK)
    mesh = plsc.VectorSubcoreMesh(core_axis_name="c", subcore_axis_name="s")
    return pl.kernel(
        _sc_body,
        out_type=jax.ShapeDtypeStruct((B_SC, DIM), jnp.float32),
        mesh=mesh,
        scratch_types=[
            pltpu.VMEM((N_CHUNKS, CHUNK), jnp.int32),
            *[pltpu.VMEM((CHUNK, DIM), jnp.float32) for _ in range(NBUF)],
            pltpu.VMEM_SHARED((VOCAB, DIM), jnp.float32),
            *[pltpu.SemaphoreType.DMA for _ in range(2 * NBUF)],
        ],
    )(table, idx2d)


def _tc_body(idx_ref, table_ref, out_ref):
    idx = idx_ref[0]
    onehot = jnp.where(
        lax.broadcasted_iota(jnp.int32, (TC_BLOCK, VPAD), 1)
        == idx.reshape(TC_BLOCK, 1),
        1.0,
        0.0,
    ).astype(jnp.float32)
    out_ref[...] = jnp.dot(
        onehot,
        table_ref[...],
        preferred_element_type=jnp.float32,
        precision=lax.Precision.HIGHEST,
    )


def _tc_part(idx_flat, table):
    idx3d = idx_flat.reshape(N_TC_BLOCKS, 1, TC_BLOCK)
    table_pad = jnp.pad(table, ((0, VPAD - VOCAB), (0, 0)))
    return pl.pallas_call(
        _tc_body,
        grid=(N_TC_BLOCKS,),
        in_specs=[
            pl.BlockSpec((1, 1, TC_BLOCK), lambda i: (i, 0, 0)),
            pl.BlockSpec((VPAD, DIM), lambda i: (0, 0)),
        ],
        out_specs=pl.BlockSpec((TC_BLOCK, DIM), lambda i: (i, 0)),
        out_shape=jax.ShapeDtypeStruct((B_TC, DIM), jnp.float32),
    )(idx3d, table_pad)


def kernel(substructure_indices, embedding_table):
    idx = substructure_indices.astype(jnp.int32).reshape(-1)
    out_sc = _sc_part(idx[:B_SC], embedding_table)
    out_tc = _tc_part(idx[B_SC:], embedding_table)
    return jnp.concatenate([out_sc, out_tc], axis=0).reshape(BATCH, SEQ, DIM)


# speedup vs baseline: 3.9754x; 2.6472x over previous
import jax
import jax.numpy as jnp
from jax import lax
from jax.experimental import pallas as pl
from jax.experimental.pallas import tpu as pltpu
from jax.experimental.pallas import tpu_sc as plsc

NC = 2
NS = 16
NW = NC * NS

VOCAB = 194
DIM = 128
B_TOTAL = 4096 * 200
B_PER_W = B_TOTAL // NW
CHUNK = 80
N_CHUNKS = B_PER_W // CHUNK
NBUF = 8
G = 4

_STEADY_LO = NBUF - G
_N_GROUPS = (N_CHUNKS - NBUF) // NBUF
_STEADY_HI = _STEADY_LO + _N_GROUPS * NBUF


def _body(table_hbm, idx_hbm, out_hbm, idx_v, *rest):
    rows = rest[:NBUF]
    table_sh = rest[NBUF]
    gsem = rest[NBUF + 1:2 * NBUF + 1]
    osem = rest[2 * NBUF + 1:]

    cid = lax.axis_index("c")
    sid = lax.axis_index("s")
    wid = sid * NC + cid
    out_base = wid * B_PER_W

    @pl.when(sid == 0)
    def _():
        pltpu.sync_copy(table_hbm, table_sh)

    plsc.subcore_barrier()

    pltpu.sync_copy(idx_hbm.at[pl.ds(wid * N_CHUNKS, N_CHUNKS)], idx_v)

    def start_gather(j, b):
        pltpu.make_async_copy(table_sh.at[idx_v.at[j]], rows[b], gsem[b]).start()

    def wait_gather(j, b):
        pltpu.make_async_copy(table_sh.at[idx_v.at[j]], rows[b], gsem[b]).wait()

    def start_out(j, b):
        pltpu.make_async_copy(
            rows[b], out_hbm.at[pl.ds(out_base + j * CHUNK, CHUNK)], osem[b]
        ).start()

    def wait_out(j, b):
        pltpu.make_async_copy(
            rows[b], out_hbm.at[pl.ds(out_base + j * CHUNK, CHUNK)], osem[b]
        ).wait()

    for b in range(G):
        start_gather(b, b)
    for j in range(_STEADY_LO):
        wait_gather(j, j)
        start_out(j, j)
        start_gather(j + G, (j + G) % NBUF)

    def group(g, _):
        j0 = _STEADY_LO + g * NBUF
        for b in range(NBUF):
            j = j0 + b
            wb = (_STEADY_LO + b) % NBUF
            wait_out(j - G, b)
            start_gather(j + G, b)
            wait_gather(j, wb)
            start_out(j, wb)
        return 0

    lax.fori_loop(0, _N_GROUPS, group, 0)

    for j in range(_STEADY_HI, N_CHUNKS):
        if j + G < N_CHUNKS:
            wait_out(j - G, (j + G) % NBUF)
            start_gather(j + G, (j + G) % NBUF)
        wait_gather(j, j % NBUF)
        start_out(j, j % NBUF)
    for j in range(N_CHUNKS - NBUF, N_CHUNKS):
        wait_out(j, j % NBUF)


def kernel(substructure_indices, embedding_table):
    idx2d = substructure_indices.astype(jnp.int32).reshape(NW * N_CHUNKS, CHUNK)
    mesh = plsc.VectorSubcoreMesh(core_axis_name="c", subcore_axis_name="s")
    out = pl.kernel(
        _body,
        out_type=jax.ShapeDtypeStruct((B_TOTAL, DIM), jnp.float32),
        mesh=mesh,
        scratch_types=[
            pltpu.VMEM((N_CHUNKS, CHUNK), jnp.int32),
            *[pltpu.VMEM((CHUNK, DIM), jnp.float32) for _ in range(NBUF)],
            pltpu.VMEM_SHARED((VOCAB, DIM), jnp.float32),
            *[pltpu.SemaphoreType.DMA for _ in range(2 * NBUF)],
        ],
    )(embedding_table, idx2d)
    return out.reshape(4096, 200, DIM)


# speedup vs baseline: 3.9876x; 1.0031x over previous
import jax
import jax.numpy as jnp
from jax import lax
from jax.experimental import pallas as pl
from jax.experimental.pallas import tpu as pltpu
from jax.experimental.pallas import tpu_sc as plsc

NC = 2
NS = 16
NW = NC * NS

VOCAB = 194
DIM = 128
B_TOTAL = 4096 * 200
B_PER_W = B_TOTAL // NW
CHUNK = 80
N_CHUNKS = B_PER_W // CHUNK
NBUF = 8
G = 3

_STEADY_LO = NBUF - G
_N_GROUPS = (N_CHUNKS - NBUF) // NBUF
_STEADY_HI = _STEADY_LO + _N_GROUPS * NBUF


def _body(table_hbm, idx_hbm, out_hbm, idx_v, *rest):
    rows = rest[:NBUF]
    table_sh = rest[NBUF]
    gsem = rest[NBUF + 1:2 * NBUF + 1]
    osem = rest[2 * NBUF + 1:]

    cid = lax.axis_index("c")
    sid = lax.axis_index("s")
    wid = sid * NC + cid
    out_base = wid * B_PER_W

    @pl.when(sid == 0)
    def _():
        pltpu.sync_copy(table_hbm, table_sh)

    plsc.subcore_barrier()

    pltpu.sync_copy(idx_hbm.at[pl.ds(wid * N_CHUNKS, N_CHUNKS)], idx_v)

    def start_gather(j, b):
        pltpu.make_async_copy(table_sh.at[idx_v.at[j]], rows[b], gsem[b]).start()

    def wait_gather(j, b):
        pltpu.make_async_copy(table_sh.at[idx_v.at[j]], rows[b], gsem[b]).wait()

    def start_out(j, b):
        pltpu.make_async_copy(
            rows[b], out_hbm.at[pl.ds(out_base + j * CHUNK, CHUNK)], osem[b]
        ).start()

    def wait_out(j, b):
        pltpu.make_async_copy(
            rows[b], out_hbm.at[pl.ds(out_base + j * CHUNK, CHUNK)], osem[b]
        ).wait()

    for b in range(G):
        start_gather(b, b)
    for j in range(_STEADY_LO):
        wait_gather(j, j)
        start_out(j, j)
        start_gather(j + G, (j + G) % NBUF)

    def group(g, _):
        j0 = _STEADY_LO + g * NBUF
        for b in range(NBUF):
            j = j0 + b
            wb = (_STEADY_LO + b) % NBUF
            wait_out(j - G, b)
            start_gather(j + G, b)
            wait_gather(j, wb)
            start_out(j, wb)
        return 0

    lax.fori_loop(0, _N_GROUPS, group, 0)

    for j in range(_STEADY_HI, N_CHUNKS):
        if j + G < N_CHUNKS:
            wait_out(j - G, (j + G) % NBUF)
            start_gather(j + G, (j + G) % NBUF)
        wait_gather(j, j % NBUF)
        start_out(j, j % NBUF)
    for j in range(N_CHUNKS - NBUF, N_CHUNKS):
        wait_out(j, j % NBUF)


def kernel(substructure_indices, embedding_table):
    idx2d = substructure_indices.astype(jnp.int32).reshape(NW * N_CHUNKS, CHUNK)
    mesh = plsc.VectorSubcoreMesh(core_axis_name="c", subcore_axis_name="s")
    out = pl.kernel(
        _body,
        out_type=jax.ShapeDtypeStruct((B_TOTAL, DIM), jnp.float32),
        mesh=mesh,
        scratch_types=[
            pltpu.VMEM((N_CHUNKS, CHUNK), jnp.int32),
            *[pltpu.VMEM((CHUNK, DIM), jnp.float32) for _ in range(NBUF)],
            pltpu.VMEM_SHARED((VOCAB, DIM), jnp.float32),
            *[pltpu.SemaphoreType.DMA for _ in range(2 * NBUF)],
        ],
    )(embedding_table, idx2d)
    return out.reshape(4096, 200, DIM)
